# baseline (speedup 1.0000x reference)
import jax
import jax.numpy as jnp
from jax import lax
from jax.experimental import pallas as pl
from jax.experimental.pallas import tpu as pltpu
from jax.experimental.pallas import tpu_sc as plsc

N = 10000
E_TOT = 320000
D_IN = 128
D_H = 64
D_PAD = 128
N_PAD = 10240
NC = 2
NS = 16
HALF = N_PAD // NC
DUMP = HALF
ACC_ROWS = HALF + 256
ACC_PER_TILE = ACC_ROWS // NS
OUT_PER_TILE = HALF // NS
CHUNK = 112
NCHUNK = 180
E_PAD = NS * NCHUNK * CHUNK
_EB = E_PAD // 128


def _dotT(x, w):
    return lax.dot_general(x, w, (((1,), (1,)), ((), ())),
                           preferred_element_type=jnp.float32)


def _make_seg_sum():
    mesh = plsc.VectorSubcoreMesh(core_axis_name="c", subcore_axis_name="s")
    out_type = [jax.ShapeDtypeStruct((N_PAD, D_PAD), jnp.float32)]
    scratch = [
        pltpu.VMEM((NCHUNK, CHUNK), jnp.int32),
        pltpu.VMEM((NCHUNK * CHUNK,), jnp.int32),
        pltpu.VMEM((CHUNK, D_PAD), jnp.float32),
        pltpu.VMEM((CHUNK, D_PAD), jnp.float32),
        pltpu.VMEM_SHARED((ACC_ROWS, D_PAD), jnp.float32),
    ] + [pltpu.SemaphoreType.DMA] * 2

    def body(y_hbm, src_hbm, rdst_hbm, z2_hbm, out_hbm,
             src_v, dst_v, b0, b1, acc_sh, gs0, gs1):
        bufs = (b0, b1)
        gsem = (gs0, gs1)
        c = lax.axis_index("c")
        s = lax.axis_index("s")
        pltpu.sync_copy(src_hbm.at[s], src_v)
        pltpu.sync_copy(rdst_hbm.at[c, s], dst_v)
        a0 = s * ACC_PER_TILE
        pltpu.sync_copy(z2_hbm, acc_sh.at[pl.ds(a0, ACC_PER_TILE)])
        plsc.subcore_barrier()

        def fire_g(j, b):
            pltpu.async_copy(y_hbm.at[src_v.at[j]], bufs[b], gsem[b])

        def wait_g(j, b):
            pltpu.make_async_copy(y_hbm.at[src_v.at[j]], bufs[b],
                                  gsem[b]).wait()

        def scat(j, b):
            pltpu.sync_copy(bufs[b],
                            acc_sh.at[dst_v.at[pl.ds(j * CHUNK, CHUNK)]],
                            add=True)

        fire_g(0, 0)
        fire_g(1, 1)

        def loop_body(t, carry):
            j0 = 2 * t
            wait_g(j0, 0)
            scat(j0, 0)
            fire_g(j0 + 2, 0)
            wait_g(j0 + 1, 1)
            scat(j0 + 1, 1)
            fire_g(j0 + 3, 1)
            return carry

        lax.fori_loop(0, NCHUNK // 2 - 1, loop_body, 0)
        wait_g(NCHUNK - 2, 0)
        scat(NCHUNK - 2, 0)
        wait_g(NCHUNK - 1, 1)
        scat(NCHUNK - 1, 1)

        plsc.subcore_barrier()
        o0 = s * OUT_PER_TILE
        pltpu.sync_copy(acc_sh.at[pl.ds(o0, OUT_PER_TILE)],
                        out_hbm.at[pl.ds(c * HALF + o0, OUT_PER_TILE)])

    return pl.kernel(body, mesh=mesh, out_type=out_type, scratch_types=scratch)


_R = 2000


def _remap_dst(dst2d):
    def body(d_ref, r0_ref, r1_ref):
        d = d_ref[...]
        r0_ref[...] = jnp.where(d < HALF, d, DUMP)
        r1_ref[...] = jnp.where(d >= HALF, d - HALF, DUMP)

    blk = pl.BlockSpec((_EB, 128), lambda: (0, 0))
    return pl.pallas_call(
        body,
        in_specs=[blk],
        out_specs=[blk, blk],
        out_shape=[jax.ShapeDtypeStruct((_EB, 128), jnp.int32)] * 2,
    )(dst2d)


def _with_count_col(y):
    r, _ = y.shape
    pad = jnp.concatenate(
        [jnp.ones((r, 1), jnp.float32),
         jnp.zeros((r, D_PAD - D_H - 1), jnp.float32)], axis=1)
    return jnp.concatenate([y, pad], axis=1)


_DB = _EB // (N // _R)


def _stage_a(X, Wl1, Wr1, dst2d):
    def body(x_ref, wl_ref, wr_ref, d_ref, xw_ref, y_ref, r0_ref, r1_ref):
        x = x_ref[...]
        xw_ref[...] = _dotT(x, wl_ref[...])
        y_ref[...] = _with_count_col(_dotT(x, wr_ref[...]))
        d = d_ref[...]
        r0_ref[...] = jnp.where(d < HALF, d, DUMP)
        r1_ref[...] = jnp.where(d >= HALF, d - HALF, DUMP)

    dblk = pl.BlockSpec((_DB, 128), lambda i: (i, 0))
    return pl.pallas_call(
        body,
        grid=(N // _R,),
        in_specs=[pl.BlockSpec((_R, D_IN), lambda i: (i, 0)),
                  pl.BlockSpec((D_H, D_IN), lambda i: (0, 0)),
                  pl.BlockSpec((D_H, D_IN), lambda i: (0, 0)),
                  dblk],
        out_specs=[pl.BlockSpec((_R, D_H), lambda i: (i, 0)),
                   pl.BlockSpec((_R, D_PAD), lambda i: (i, 0)),
                   dblk, dblk],
        out_shape=[jax.ShapeDtypeStruct((N, D_H), jnp.float32),
                   jax.ShapeDtypeStruct((N_PAD, D_PAD), jnp.float32),
                   jax.ShapeDtypeStruct((_EB, 128), jnp.int32),
                   jax.ShapeDtypeStruct((_EB, 128), jnp.int32)],
    )(X, Wl1, Wr1, dst2d)


def _sage_finish(xw, ssum, cnt, b):
    out = xw + ssum / jnp.maximum(cnt, 1.0) + b
    nrm = jnp.sqrt(jnp.sum(out * out, axis=1, keepdims=True))
    return jnp.maximum(out / jnp.maximum(nrm, 1e-12), 0.0)


def _sage_finish_p(xw, p, b):
    cnt = jnp.maximum(p[:, D_H:D_H + 1], 1.0)
    out = xw + p[:, :D_H] / cnt + b
    nrm = jnp.sqrt(jnp.sum(out * out, axis=1, keepdims=True))
    return jnp.maximum(out / jnp.maximum(nrm, 1e-12), 0.0), cnt


def _stage_b(XW1, P1, b1, Wl2, Wr2):
    def body(xw_ref, p_ref, b_ref, wl_ref, wr_ref, xw2_ref, y2_ref, c_ref):
        h, cnt = _sage_finish_p(xw_ref[...], p_ref[...], b_ref[...])
        c_ref[...] = cnt
        xw2_ref[...] = _dotT(h, wl_ref[...])
        y2_ref[...] = _with_count_col(_dotT(h, wr_ref[...]))

    row = pl.BlockSpec((_R, D_H), lambda i: (i, 0))
    wide = pl.BlockSpec((_R, D_PAD), lambda i: (i, 0))
    cst = pl.BlockSpec((1, D_H), lambda i: (0, 0))
    wsp = pl.BlockSpec((D_H, D_H), lambda i: (0, 0))
    return pl.pallas_call(
        body,
        grid=(N // _R,),
        in_specs=[row, wide, cst, wsp, wsp],
        out_specs=[row, wide, pl.BlockSpec((_R, 1), lambda i: (i, 0))],
        out_shape=[jax.ShapeDtypeStruct((N, D_H), jnp.float32),
                   jax.ShapeDtypeStruct((N_PAD, D_PAD), jnp.float32),
                   jax.ShapeDtypeStruct((N, 1), jnp.float32)],
    )(XW1, P1, b1, Wl2, Wr2)


def _stage_c(XW2, P2, C, b2, Wp1, bp1, Wp2, bp2):
    def body(xw_ref, p_ref, c_ref, b_ref, wp1_ref, bp1_ref, wp2_ref, bp2_ref,
             o_ref):
        p = p_ref[...]
        out = xw_ref[...] + p[:, :D_H] / c_ref[...] + b_ref[...]
        nrm = jnp.sqrt(jnp.sum(out * out, axis=1, keepdims=True))
        h = jnp.maximum(out / jnp.maximum(nrm, 1e-12), 0.0)
        h = _dotT(h, wp1_ref[...]) + bp1_ref[...]
        o_ref[...] = _dotT(h, wp2_ref[...]) + bp2_ref[...]

    row = pl.BlockSpec((_R, D_H), lambda i: (i, 0))
    wide = pl.BlockSpec((_R, D_PAD), lambda i: (i, 0))
    col = pl.BlockSpec((_R, 1), lambda i: (i, 0))
    cst = pl.BlockSpec((1, D_H), lambda i: (0, 0))
    wsp = pl.BlockSpec((D_H, D_H), lambda i: (0, 0))
    return pl.pallas_call(
        body,
        grid=(N // _R,),
        in_specs=[row, wide, col, cst, wsp, cst, wsp, cst],
        out_specs=row,
        out_shape=jax.ShapeDtypeStruct((N, D_H), jnp.float32),
    )(XW2, P2, C, b2, Wp1, bp1, Wp2, bp2)


def kernel(X, E, Wl1, bl1, Wr1, br1, Wl2, bl2, Wr2, br2, Wp1, bp1, Wp2, bp2):
    npad = E_PAD - E_TOT
    src = jnp.concatenate(
        [E[0].astype(jnp.int32), jnp.zeros((npad,), jnp.int32)]
    ).reshape(NS, NCHUNK, CHUNK)
    dpad = jnp.concatenate(
        [E[1].astype(jnp.int32), jnp.full((npad,), N_PAD, jnp.int32)])
    r0, r1 = _remap_dst(dpad.reshape(_EB, 128))
    rdst = jnp.stack([r0.reshape(NS, NCHUNK * CHUNK),
                      r1.reshape(NS, NCHUNK * CHUNK)])
    z2 = jnp.zeros((ACC_PER_TILE, D_PAD), jnp.float32)
    b1 = (bl1 + br1).reshape(1, D_H)
    b2 = (bl2 + br2).reshape(1, D_H)

    XW1, Y1, r0, r1 = _stage_a(X, Wl1, Wr1, dpad.reshape(_EB, 128))
    rdst = jnp.stack([r0.reshape(NS, NCHUNK * CHUNK),
                      r1.reshape(NS, NCHUNK * CHUNK)])

    seg = _make_seg_sum()
    P1 = seg(Y1, src, rdst, z2)
    if isinstance(P1, (list, tuple)):
        P1 = P1[0]

    XW2, Y2, C = _stage_b(XW1, P1, b1, Wl2, Wr2)

    P2 = seg(Y2, src, rdst, z2)
    if isinstance(P2, (list, tuple)):
        P2 = P2[0]

    return _stage_c(XW2, P2, C, b2,
                    Wp1, bp1.reshape(1, D_H), Wp2, bp2.reshape(1, D_H))


# speedup vs baseline: 1.5951x; 1.5951x over previous
import jax
import jax.numpy as jnp
from jax import lax
from jax.experimental import pallas as pl
from jax.experimental.pallas import tpu as pltpu
from jax.experimental.pallas import tpu_sc as plsc

N = 10000
E_TOT = 320000
D_IN = 128
D_H = 64
D_PAD = 128
N_PAD = 10240
NC = 2
NS = 16
HALF = N_PAD // NC
DUMP = HALF
ACC_ROWS = HALF + 256
ACC_PER_TILE = ACC_ROWS // NS
OUT_PER_TILE = HALF // NS
CHUNK = 80
NCHUNK = 250
E_PAD = NS * NCHUNK * CHUNK
_EB = E_PAD // 128


def _dotT(x, w):
    return lax.dot_general(x, w, (((1,), (1,)), ((), ())),
                           preferred_element_type=jnp.float32)


def _make_seg_sum():
    mesh = plsc.VectorSubcoreMesh(core_axis_name="c", subcore_axis_name="s")
    out_type = [jax.ShapeDtypeStruct((N_PAD, D_PAD), jnp.float32)]
    scratch = [
        pltpu.VMEM((NCHUNK, CHUNK), jnp.int32),
        pltpu.VMEM((NCHUNK * CHUNK,), jnp.int32),
        pltpu.VMEM((CHUNK, D_PAD), jnp.float32),
        pltpu.VMEM((CHUNK, D_PAD), jnp.float32),
        pltpu.VMEM_SHARED((ACC_ROWS, D_PAD), jnp.float32),
    ] + [pltpu.SemaphoreType.DMA] * 2

    def body(y_hbm, src_hbm, rdst_hbm, z2_hbm, out_hbm,
             src_v, dst_v, b0, b1, acc_sh, gs0, gs1):
        bufs = (b0, b1)
        gsem = (gs0, gs1)
        c = lax.axis_index("c")
        s = lax.axis_index("s")
        pltpu.sync_copy(src_hbm.at[s], src_v)
        pltpu.sync_copy(rdst_hbm.at[c, s], dst_v)
        a0 = s * ACC_PER_TILE
        pltpu.sync_copy(z2_hbm.at[pl.ds(a0, ACC_PER_TILE)],
                        acc_sh.at[pl.ds(a0, ACC_PER_TILE)])
        plsc.subcore_barrier()

        def fire_g(j, b):
            pltpu.async_copy(y_hbm.at[src_v.at[j]], bufs[b], gsem[b])

        def wait_g(j, b):
            pltpu.make_async_copy(y_hbm.at[src_v.at[j]], bufs[b],
                                  gsem[b]).wait()

        def scat(j, b):
            pltpu.sync_copy(bufs[b],
                            acc_sh.at[dst_v.at[pl.ds(j * CHUNK, CHUNK)]],
                            add=True)

        fire_g(0, 0)
        fire_g(1, 1)

        def loop_body(t, carry):
            j0 = 2 * t
            wait_g(j0, 0)
            scat(j0, 0)
            fire_g(j0 + 2, 0)
            wait_g(j0 + 1, 1)
            scat(j0 + 1, 1)
            fire_g(j0 + 3, 1)
            return carry

        lax.fori_loop(0, NCHUNK // 2 - 1, loop_body, 0)
        wait_g(NCHUNK - 2, 0)
        scat(NCHUNK - 2, 0)
        wait_g(NCHUNK - 1, 1)
        scat(NCHUNK - 1, 1)

        plsc.subcore_barrier()
        o0 = s * OUT_PER_TILE
        pltpu.sync_copy(acc_sh.at[pl.ds(o0, OUT_PER_TILE)],
                        out_hbm.at[pl.ds(c * HALF + o0, OUT_PER_TILE)])

    return pl.kernel(body, mesh=mesh, out_type=out_type, scratch_types=scratch)


_R = 2000


def _remap_dst(dst2d):
    def body(d_ref, r0_ref, r1_ref):
        d = d_ref[...]
        r0_ref[...] = jnp.where(d < HALF, d, DUMP)
        r1_ref[...] = jnp.where(d >= HALF, d - HALF, DUMP)

    blk = pl.BlockSpec((_EB, 128), lambda: (0, 0))
    return pl.pallas_call(
        body,
        in_specs=[blk],
        out_specs=[blk, blk],
        out_shape=[jax.ShapeDtypeStruct((_EB, 128), jnp.int32)] * 2,
    )(dst2d)


def _with_count_col(y):
    r, _ = y.shape
    pad = jnp.concatenate(
        [jnp.ones((r, 1), jnp.float32),
         jnp.zeros((r, D_PAD - D_H - 1), jnp.float32)], axis=1)
    return jnp.concatenate([y, pad], axis=1)


def _stage_a(X, Wl1, Wr1):
    def body(x_ref, wl_ref, wr_ref, xw_ref, y_ref):
        x = x_ref[...]
        xw_ref[...] = _dotT(x, wl_ref[...])
        y_ref[...] = _with_count_col(_dotT(x, wr_ref[...]))

    return pl.pallas_call(
        body,
        grid=(N // _R,),
        in_specs=[pl.BlockSpec((_R, D_IN), lambda i: (i, 0)),
                  pl.BlockSpec((D_H, D_IN), lambda i: (0, 0)),
                  pl.BlockSpec((D_H, D_IN), lambda i: (0, 0))],
        out_specs=[pl.BlockSpec((_R, D_H), lambda i: (i, 0)),
                   pl.BlockSpec((_R, D_PAD), lambda i: (i, 0))],
        out_shape=[jax.ShapeDtypeStruct((N, D_H), jnp.float32),
                   jax.ShapeDtypeStruct((N_PAD, D_PAD), jnp.float32)],
    )(X, Wl1, Wr1)


def _sage_finish(xw, ssum, cnt, b):
    out = xw + ssum / jnp.maximum(cnt, 1.0) + b
    nrm = jnp.sqrt(jnp.sum(out * out, axis=1, keepdims=True))
    return jnp.maximum(out / jnp.maximum(nrm, 1e-12), 0.0)


def _sage_finish_p(xw, p, b):
    cnt = jnp.maximum(p[:, D_H:D_H + 1], 1.0)
    out = xw + p[:, :D_H] / cnt + b
    nrm = jnp.sqrt(jnp.sum(out * out, axis=1, keepdims=True))
    return jnp.maximum(out / jnp.maximum(nrm, 1e-12), 0.0), cnt


def _stage_b(XW1, P1, b1, Wl2, Wr2):
    def body(xw_ref, p_ref, b_ref, wl_ref, wr_ref, xw2_ref, y2_ref, c_ref):
        h, cnt = _sage_finish_p(xw_ref[...], p_ref[...], b_ref[...])
        c_ref[...] = cnt
        xw2_ref[...] = _dotT(h, wl_ref[...])
        y2_ref[...] = _with_count_col(_dotT(h, wr_ref[...]))

    row = pl.BlockSpec((_R, D_H), lambda i: (i, 0))
    wide = pl.BlockSpec((_R, D_PAD), lambda i: (i, 0))
    cst = pl.BlockSpec((1, D_H), lambda i: (0, 0))
    wsp = pl.BlockSpec((D_H, D_H), lambda i: (0, 0))
    return pl.pallas_call(
        body,
        grid=(N // _R,),
        in_specs=[row, wide, cst, wsp, wsp],
        out_specs=[row, wide, pl.BlockSpec((_R, 1), lambda i: (i, 0))],
        out_shape=[jax.ShapeDtypeStruct((N, D_H), jnp.float32),
                   jax.ShapeDtypeStruct((N_PAD, D_PAD), jnp.float32),
                   jax.ShapeDtypeStruct((N, 1), jnp.float32)],
    )(XW1, P1, b1, Wl2, Wr2)


def _stage_c(XW2, P2, C, b2, Wp1, bp1, Wp2, bp2):
    def body(xw_ref, p_ref, c_ref, b_ref, wp1_ref, bp1_ref, wp2_ref, bp2_ref,
             o_ref):
        p = p_ref[...]
        out = xw_ref[...] + p[:, :D_H] / c_ref[...] + b_ref[...]
        nrm = jnp.sqrt(jnp.sum(out * out, axis=1, keepdims=True))
        h = jnp.maximum(out / jnp.maximum(nrm, 1e-12), 0.0)
        h = _dotT(h, wp1_ref[...]) + bp1_ref[...]
        o_ref[...] = _dotT(h, wp2_ref[...]) + bp2_ref[...]

    row = pl.BlockSpec((_R, D_H), lambda i: (i, 0))
    wide = pl.BlockSpec((_R, D_PAD), lambda i: (i, 0))
    col = pl.BlockSpec((_R, 1), lambda i: (i, 0))
    cst = pl.BlockSpec((1, D_H), lambda i: (0, 0))
    wsp = pl.BlockSpec((D_H, D_H), lambda i: (0, 0))
    return pl.pallas_call(
        body,
        grid=(N // _R,),
        in_specs=[row, wide, col, cst, wsp, cst, wsp, cst],
        out_specs=row,
        out_shape=jax.ShapeDtypeStruct((N, D_H), jnp.float32),
    )(XW2, P2, C, b2, Wp1, bp1, Wp2, bp2)


def kernel(X, E, Wl1, bl1, Wr1, br1, Wl2, bl2, Wr2, br2, Wp1, bp1, Wp2, bp2):
    npad = E_PAD - E_TOT
    src = jnp.concatenate(
        [E[0].astype(jnp.int32), jnp.zeros((npad,), jnp.int32)]
    ).reshape(NS, NCHUNK, CHUNK)
    dpad = jnp.concatenate(
        [E[1].astype(jnp.int32), jnp.full((npad,), N_PAD, jnp.int32)])
    r0, r1 = _remap_dst(dpad.reshape(_EB, 128))
    rdst = jnp.stack([r0.reshape(NS, NCHUNK * CHUNK),
                      r1.reshape(NS, NCHUNK * CHUNK)])
    z2 = jnp.zeros((ACC_ROWS, D_PAD), jnp.float32)
    b1 = (bl1 + br1).reshape(1, D_H)
    b2 = (bl2 + br2).reshape(1, D_H)

    XW1, Y1 = _stage_a(X, Wl1, Wr1)

    seg = _make_seg_sum()
    P1 = seg(Y1, src, rdst, z2)
    if isinstance(P1, (list, tuple)):
        P1 = P1[0]

    XW2, Y2, C = _stage_b(XW1, P1, b1, Wl2, Wr2)

    P2 = seg(Y2, src, rdst, z2)
    if isinstance(P2, (list, tuple)):
        P2 = P2[0]

    return _stage_c(XW2, P2, C, b2,
                    Wp1, bp1.reshape(1, D_H), Wp2, bp2.reshape(1, D_H))
